# baseline (speedup 1.0000x reference)
import functools
import jax
import jax.numpy as jnp
from jax import lax
from jax.experimental import pallas as pl
from jax.experimental.pallas import tpu as pltpu
from jax.experimental.pallas import tpu_sc as plsc

NUM_DEVICE_TYPES = 100000
EMBED_DIM = 64
BATCH = 4096
HIST_LEN = 50
SCALE = 8.0

NC = 2
NS = 16
NW = NC * NS
TOTAL = BATCH * HIST_LEN
CHUNK = 128
NCHUNK_ALL = TOTAL // CHUNK
NCHUNK = NCHUNK_ALL // NW
BBLOCKS = BATCH // CHUNK
NBUF_G = 4
NBUF_T = 3
LOOKAHEAD = NBUF_G - 1


@functools.partial(
    pl.kernel,
    mesh=plsc.VectorSubcoreMesh(core_axis_name="c", subcore_axis_name="s"),
    out_type=jax.ShapeDtypeStruct((HIST_LEN, EMBED_DIM, BATCH), jnp.float32),
    scratch_types=[
        pltpu.VMEM((NCHUNK, CHUNK), jnp.int32),
        pltpu.VMEM((NBUF_G, CHUNK, EMBED_DIM), jnp.float32),
        pltpu.VMEM((NBUF_T, EMBED_DIM, CHUNK), jnp.float32),
        pltpu.SemaphoreType.DMA((NBUF_G,)),
        pltpu.SemaphoreType.DMA((NBUF_T,)),
    ],
    compiler_params=pltpu.CompilerParams(use_tc_tiling_on_sc=False,
                                         needs_layout_passes=False),
)
def _embed_gather(table_hbm, idx_hbm, out_hbm, idx_v, rows_v, trows_v,
                  gsem, osem):
    wid = lax.axis_index("s") * NC + lax.axis_index("c")
    cbase = wid * NCHUNK
    pltpu.sync_copy(idx_hbm.at[pl.ds(cbase, NCHUNK)], idx_v)

    lane = lax.iota(jnp.int32, 16)

    def gather(j, gb):
        return pltpu.make_async_copy(
            table_hbm.at[idx_v.at[j]], rows_v.at[gb], gsem.at[gb])

    def writeback(j, tb):
        flat = cbase + j
        h = flat // BBLOCKS
        b0 = (flat % BBLOCKS) * CHUNK
        return pltpu.make_async_copy(
            trows_v.at[tb], out_hbm.at[h, :, pl.ds(b0, CHUNK)], osem.at[tb])

    for g in range(LOOKAHEAD):
        gather(g, g).start()

    def chunk_body(j, carry):
        gb = j % NBUF_G
        tb = j % NBUF_T

        @pl.when(j + LOOKAHEAD < NCHUNK)
        def _():
            gather(j + LOOKAHEAD, (j + LOOKAHEAD) % NBUF_G).start()

        gather(j, gb).wait()

        @pl.when(j >= NBUF_T)
        def _():
            writeback(j - NBUF_T, tb).wait()

        rv = rows_v.at[gb]
        tv = trows_v.at[tb]

        def dloop(d, c2):
            dvec = jnp.full((16,), 0, jnp.int32) + d
            for k in range(CHUNK // 16):
                ridx = lane + (16 * k)
                vals = plsc.load_gather(rv, [ridx, dvec])
                tv[d, pl.ds(16 * k, 16)] = vals * SCALE
            return c2

        lax.fori_loop(0, EMBED_DIM, dloop, 0)
        writeback(j, tb).start()
        return carry

    lax.fori_loop(0, NCHUNK, chunk_body, 0)

    for j in range(NCHUNK - NBUF_T, NCHUNK):
        writeback(j, j % NBUF_T).wait()


def kernel(x, table):
    idx = x.astype(jnp.int32).T.reshape(NCHUNK_ALL, CHUNK)
    out = _embed_gather(table, idx)
    return out.transpose(2, 0, 1)


# speedup vs baseline: 1.2518x; 1.2518x over previous
import functools
import jax
import jax.numpy as jnp
from jax import lax
from jax.experimental import pallas as pl
from jax.experimental.pallas import tpu as pltpu
from jax.experimental.pallas import tpu_sc as plsc

NUM_DEVICE_TYPES = 100000
EMBED_DIM = 64
BATCH = 4096
HIST_LEN = 50
SCALE = 8.0

NC = 2
NS = 16
NW = NC * NS
NROUND = EMBED_DIM // NW
UNROLL = 8


@functools.partial(
    pl.kernel,
    mesh=plsc.VectorSubcoreMesh(core_axis_name="c", subcore_axis_name="s"),
    out_type=jax.ShapeDtypeStruct((HIST_LEN, EMBED_DIM, BATCH), jnp.float32),
    scratch_types=[
        pltpu.VMEM_SHARED((HIST_LEN, BATCH), jnp.int32),
        pltpu.VMEM((NUM_DEVICE_TYPES,), jnp.float32),
        pltpu.VMEM((2, BATCH), jnp.int32),
        pltpu.VMEM((2, BATCH), jnp.float32),
        pltpu.SemaphoreType.DMA,
        pltpu.SemaphoreType.DMA((2,)),
        pltpu.SemaphoreType.DMA((2,)),
    ],
    compiler_params=pltpu.CompilerParams(use_tc_tiling_on_sc=False,
                                         needs_layout_passes=False),
)
def _embed_gather(tableT_hbm, idxT_hbm, out_hbm, idx_sh, trow_v, icol_v,
                  ocol_v, tsem, isem, osem):
    cid = lax.axis_index("c")
    sid = lax.axis_index("s")
    wid = sid * NC + cid

    @pl.when(sid == 0)
    def _():
        pltpu.sync_copy(idxT_hbm, idx_sh)

    plsc.subcore_barrier()

    def icol_copy(h, ib):
        return pltpu.make_async_copy(idx_sh.at[h], icol_v.at[ib], isem.at[ib])

    def trow_copy(d):
        return pltpu.make_async_copy(tableT_hbm.at[d], trow_v, tsem)

    def ocol_copy(h, d, ob):
        return pltpu.make_async_copy(ocol_v.at[ob], out_hbm.at[h, d],
                                     osem.at[ob])

    for rnd in range(NROUND):
        d = rnd * NW + wid
        trow_copy(d).start()
        icol_copy(0, 0).start()
        trow_copy(d).wait()

        def col_body(h, carry):
            ib = h % 2

            @pl.when(h + 1 < HIST_LEN)
            def _():
                icol_copy(h + 1, (h + 1) % 2).start()

            icol_copy(h, ib).wait()

            @pl.when(h >= 2)
            def _():
                ocol_copy(h - 2, d, ib).wait()

            icol = icol_v.at[ib]
            ocol = ocol_v.at[ib]

            def gloop(q, c2):
                base = q * (16 * UNROLL)
                for u in range(UNROLL):
                    sl = pl.ds(base + u * 16, 16)
                    vals = plsc.load_gather(trow_v, [icol[sl]])
                    ocol[sl] = vals * SCALE
                return c2

            lax.fori_loop(0, BATCH // (16 * UNROLL), gloop, 0)
            ocol_copy(h, d, ib).start()
            return carry

        lax.fori_loop(0, HIST_LEN, col_body, 0)

        for h in range(HIST_LEN - 2, HIST_LEN):
            ocol_copy(h, d, h % 2).wait()


def kernel(x, table):
    idxT = x.astype(jnp.int32).T
    tableT = table.T
    out = _embed_gather(tableT, idxT)
    return out.transpose(2, 0, 1)
